# baseline (speedup 1.0000x reference)
import functools

import jax
import jax.numpy as jnp
from jax import lax
from jax.experimental import pallas as pl
from jax.experimental.pallas import tpu as pltpu
from jax.experimental.pallas import tpu_sc as plsc

NUM_ITEMS = 8192
BATCH = 1024
ENC_H = 600
Z_DIM = 200
TAU = 0.2
EPS = 1e-12

_NC = 2
_NS = 16
_NW = _NC * _NS
_ROWS_PER_W = BATCH // _NW
_CH = 8
_NCHUNK = _ROWS_PER_W // _CH

_BM = 128
_NBLK = BATCH // _BM


def _sc_gather(gram, idx2d):
    mesh = plsc.VectorSubcoreMesh(core_axis_name="c", subcore_axis_name="s")

    @functools.partial(
        pl.kernel,
        mesh=mesh,
        out_type=jax.ShapeDtypeStruct((BATCH, NUM_ITEMS), jnp.float32),
        scratch_types=[
            pltpu.VMEM((_NCHUNK, _CH), jnp.int32),
            pltpu.VMEM((_CH, NUM_ITEMS), jnp.float32),
            pltpu.SemaphoreType.DMA,
        ],
    )
    def gather_kernel(table_hbm, idx_hbm, out_hbm, idx_v, buf, sem):
        wid = lax.axis_index("s") * _NC + lax.axis_index("c")
        base = wid * _ROWS_PER_W
        pltpu.sync_copy(idx_hbm.at[pl.ds(wid * _NCHUNK, _NCHUNK)], idx_v)
        for c in range(_NCHUNK):
            pltpu.async_copy(table_hbm.at[idx_v.at[c]], buf, sem).wait()
            pltpu.sync_copy(buf, out_hbm.at[pl.ds(base + c * _CH, _CH)])

    return gather_kernel(gram, idx2d)


def _prep_body(w1_ref, itemsT_ref, w1b_ref, itn_ref):
    w1b_ref[...] = w1_ref[...].astype(jnp.bfloat16)
    it = itemsT_ref[...]
    s = jnp.sqrt(jnp.sum(it * it, axis=0, keepdims=True))
    itn_ref[...] = (it / jnp.maximum(s, EPS)).astype(jnp.bfloat16)


def _tc_prep(W1, itemsT):
    return pl.pallas_call(
        _prep_body,
        out_shape=[
            jax.ShapeDtypeStruct((NUM_ITEMS, ENC_H), jnp.bfloat16),
            jax.ShapeDtypeStruct((Z_DIM, NUM_ITEMS), jnp.bfloat16),
        ],
    )(W1, itemsT)


def _tc_body(gath_ref, rate_ref, w1b_ref, b1_ref, w2_ref, b2_ref, itn_ref,
             eps_ref, z_ref, logits_ref, klrow_ref, acc_ref):
    i = pl.program_id(0)

    vb = jnp.where(rate_ref[...] > 0, gath_ref[...], 0.0).astype(jnp.bfloat16)
    vf = vb.astype(jnp.float32)
    s1 = jnp.maximum(jnp.sum(vf, axis=1, keepdims=True), EPS)
    l2 = jnp.sqrt(jnp.sum(vf * vf, axis=1, keepdims=True)) / s1
    scale = 1.0 / (s1 * jnp.maximum(l2, EPS))

    pre = jnp.dot(vb, w1b_ref[...], preferred_element_type=jnp.float32)
    h = jnp.tanh(pre * scale + b1_ref[...])
    x2 = jnp.dot(h, w2_ref[...], preferred_element_type=jnp.float32) + b2_ref[...]
    mean = x2[:, :Z_DIM]
    logvar = x2[:, Z_DIM:]
    std = jnp.exp(0.5 * logvar)
    z = mean + eps_ref[...] * std
    z_ref[...] = z

    zn = z * ((1.0 / TAU) /
              jnp.maximum(jnp.sqrt(jnp.sum(z * z, axis=1, keepdims=True)), EPS))
    logits_ref[...] = jnp.dot(zn.astype(jnp.bfloat16), itn_ref[...],
                              preferred_element_type=jnp.float32)

    var = std * std
    klb = jnp.sum(mean * mean + var - 1.0 - logvar)
    prev = jnp.where(i == 0, 0.0, acc_ref[0, 0])
    total = prev + klb
    acc_ref[0, 0] = total
    klrow_ref[...] = jnp.full((1, 1, 128), total * (0.5 / BATCH), jnp.float32)


def _tc_encoder(gathered, rating, W1b, b1, W2, b2, itn, epsilon):
    return pl.pallas_call(
        _tc_body,
        grid=(_NBLK,),
        in_specs=[
            pl.BlockSpec((_BM, NUM_ITEMS), lambda i: (i, 0)),
            pl.BlockSpec((_BM, NUM_ITEMS), lambda i: (i, 0)),
            pl.BlockSpec((NUM_ITEMS, ENC_H), lambda i: (0, 0)),
            pl.BlockSpec((1, ENC_H), lambda i: (0, 0)),
            pl.BlockSpec((ENC_H, 2 * Z_DIM), lambda i: (0, 0)),
            pl.BlockSpec((1, 2 * Z_DIM), lambda i: (0, 0)),
            pl.BlockSpec((Z_DIM, NUM_ITEMS), lambda i: (0, 0)),
            pl.BlockSpec((_BM, Z_DIM), lambda i: (i, 0)),
        ],
        out_specs=[
            pl.BlockSpec((_BM, Z_DIM), lambda i: (i, 0)),
            pl.BlockSpec((_BM, NUM_ITEMS), lambda i: (i, 0)),
            pl.BlockSpec((1, 1, 128), lambda i: (0, 0, 0)),
        ],
        out_shape=[
            jax.ShapeDtypeStruct((BATCH, Z_DIM), jnp.float32),
            jax.ShapeDtypeStruct((BATCH, NUM_ITEMS), jnp.float32),
            jax.ShapeDtypeStruct((1, 1, 128), jnp.float32),
        ],
        scratch_shapes=[
            pltpu.SMEM((1, 1), jnp.float32),
        ],
    )(gathered, rating, W1b, b1, W2, b2, itn, epsilon)


def kernel(rating_matrix_batch, rating_matrix_batch2, gram_matrix, W1, b1, W2,
           b2, items, epsilon):
    idx2d = rating_matrix_batch2.astype(jnp.int32).reshape(BATCH // _CH, _CH)
    gathered = _sc_gather(gram_matrix, idx2d)
    W1b, itn = _tc_prep(W1, items.T)
    z, logits, klrows = _tc_encoder(
        gathered, rating_matrix_batch, W1b, b1.reshape(1, ENC_H), W2,
        b2.reshape(1, 2 * Z_DIM), itn, epsilon)
    kl = klrows[0, 0, 0]
    return z, logits, kl


# speedup vs baseline: 1.0185x; 1.0185x over previous
import functools

import jax
import jax.numpy as jnp
from jax import lax
from jax.experimental import pallas as pl
from jax.experimental.pallas import tpu as pltpu
from jax.experimental.pallas import tpu_sc as plsc

NUM_ITEMS = 8192
BATCH = 1024
ENC_H = 600
Z_DIM = 200
TAU = 0.2
EPS = 1e-12

_NC = 2
_NS = 16
_NW = _NC * _NS
_ROWS_PER_W = BATCH // _NW
_CH = 8
_NCHUNK = _ROWS_PER_W // _CH

_BM = 128
_NBLK = BATCH // _BM


def _sc_gather(gram, idx):
    mesh = plsc.VectorSubcoreMesh(core_axis_name="c", subcore_axis_name="s")

    @functools.partial(
        pl.kernel,
        mesh=mesh,
        out_type=jax.ShapeDtypeStruct((BATCH, NUM_ITEMS), jnp.float32),
        scratch_types=[
            pltpu.VMEM((_ROWS_PER_W,), jnp.int32),
            pltpu.VMEM((_CH, NUM_ITEMS), jnp.float32),
            pltpu.SemaphoreType.DMA,
        ],
    )
    def gather_kernel(table_hbm, idx_hbm, out_hbm, idx_v, buf, sem):
        wid = lax.axis_index("s") * _NC + lax.axis_index("c")
        base = wid * _ROWS_PER_W
        pltpu.sync_copy(idx_hbm.at[pl.ds(base, _ROWS_PER_W)], idx_v)
        for c in range(_NCHUNK):
            pltpu.async_copy(
                table_hbm.at[idx_v.at[pl.ds(c * _CH, _CH)]], buf, sem).wait()
            pltpu.sync_copy(buf, out_hbm.at[pl.ds(base + c * _CH, _CH)])

    return gather_kernel(gram, idx)


_PREP_BM = 128


_KCH = NUM_ITEMS // (BATCH // _PREP_BM)


def _prep_body(rate_ref, w1T_ref, itemsT_ref, mask_ref, w1b_ref, itn_ref):
    i = pl.program_id(0)
    mask_ref[...] = (rate_ref[...] > 0).astype(jnp.bfloat16)
    w1b_ref[...] = jnp.transpose(w1T_ref[...].astype(jnp.bfloat16))

    @pl.when(i == 0)
    def _():
        it = itemsT_ref[...]
        s = jnp.sqrt(jnp.sum(it * it, axis=0, keepdims=True))
        itn_ref[...] = (it / jnp.maximum(s, EPS)).astype(jnp.bfloat16)


def _tc_prep(rating, W1T, itemsT):
    return pl.pallas_call(
        _prep_body,
        grid=(BATCH // _PREP_BM,),
        in_specs=[
            pl.BlockSpec((_PREP_BM, NUM_ITEMS), lambda i: (i, 0)),
            pl.BlockSpec((ENC_H, _KCH), lambda i: (0, i)),
            pl.BlockSpec((Z_DIM, NUM_ITEMS), lambda i: (0, 0)),
        ],
        out_specs=[
            pl.BlockSpec((_PREP_BM, NUM_ITEMS), lambda i: (i, 0)),
            pl.BlockSpec((_KCH, ENC_H), lambda i: (i, 0)),
            pl.BlockSpec((Z_DIM, NUM_ITEMS), lambda i: (0, 0)),
        ],
        out_shape=[
            jax.ShapeDtypeStruct((BATCH, NUM_ITEMS), jnp.bfloat16),
            jax.ShapeDtypeStruct((NUM_ITEMS, ENC_H), jnp.bfloat16),
            jax.ShapeDtypeStruct((Z_DIM, NUM_ITEMS), jnp.bfloat16),
        ],
    )(rating, W1T, itemsT)


def _tc_body(gath_ref, rate_ref, w1b_ref, b1_ref, w2_ref, b2_ref, itn_ref,
             eps_ref, z_ref, logits_ref, klrow_ref, acc_ref):
    i = pl.program_id(0)

    vb = (gath_ref[...] * rate_ref[...].astype(jnp.float32)).astype(jnp.bfloat16)
    vf = vb.astype(jnp.float32)
    s1 = jnp.maximum(jnp.sum(vf, axis=1, keepdims=True), EPS)
    l2 = jnp.sqrt(jnp.sum(vf * vf, axis=1, keepdims=True)) / s1
    scale = 1.0 / (s1 * jnp.maximum(l2, EPS))

    pre = jnp.dot(vb, w1b_ref[...], preferred_element_type=jnp.float32)
    h = jnp.tanh(pre * scale + b1_ref[...])
    x2 = jnp.dot(h, w2_ref[...], preferred_element_type=jnp.float32) + b2_ref[...]
    mean = x2[:, :Z_DIM]
    logvar = x2[:, Z_DIM:]
    std = jnp.exp(0.5 * logvar)
    z = mean + eps_ref[...] * std
    z_ref[...] = z

    zn = z * ((1.0 / TAU) /
              jnp.maximum(jnp.sqrt(jnp.sum(z * z, axis=1, keepdims=True)), EPS))
    logits_ref[...] = jnp.dot(zn.astype(jnp.bfloat16), itn_ref[...],
                              preferred_element_type=jnp.float32)

    var = std * std
    klb = jnp.sum(mean * mean + var - 1.0 - logvar)
    prev = jnp.where(i == 0, 0.0, acc_ref[0, 0])
    total = prev + klb
    acc_ref[0, 0] = total
    klrow_ref[...] = jnp.full((1, 1, 128), total * (0.5 / BATCH), jnp.float32)


def _tc_encoder(gathered, mask, W1b, b1, W2, b2, itn, epsilon):
    return pl.pallas_call(
        _tc_body,
        grid=(_NBLK,),
        in_specs=[
            pl.BlockSpec((_BM, NUM_ITEMS), lambda i: (i, 0)),
            pl.BlockSpec((_BM, NUM_ITEMS), lambda i: (i, 0)),
            pl.BlockSpec((NUM_ITEMS, ENC_H), lambda i: (0, 0)),
            pl.BlockSpec((1, ENC_H), lambda i: (0, 0)),
            pl.BlockSpec((ENC_H, 2 * Z_DIM), lambda i: (0, 0)),
            pl.BlockSpec((1, 2 * Z_DIM), lambda i: (0, 0)),
            pl.BlockSpec((Z_DIM, NUM_ITEMS), lambda i: (0, 0)),
            pl.BlockSpec((_BM, Z_DIM), lambda i: (i, 0)),
        ],
        out_specs=[
            pl.BlockSpec((_BM, Z_DIM), lambda i: (i, 0)),
            pl.BlockSpec((_BM, NUM_ITEMS), lambda i: (i, 0)),
            pl.BlockSpec((1, 1, 128), lambda i: (0, 0, 0)),
        ],
        out_shape=[
            jax.ShapeDtypeStruct((BATCH, Z_DIM), jnp.float32),
            jax.ShapeDtypeStruct((BATCH, NUM_ITEMS), jnp.float32),
            jax.ShapeDtypeStruct((1, 1, 128), jnp.float32),
        ],
        scratch_shapes=[
            pltpu.SMEM((1, 1), jnp.float32),
        ],
    )(gathered, mask, W1b, b1, W2, b2, itn, epsilon)


def kernel(rating_matrix_batch, rating_matrix_batch2, gram_matrix, W1, b1, W2,
           b2, items, epsilon):
    gathered = _sc_gather(gram_matrix, rating_matrix_batch2)
    mask, W1b, itn = _tc_prep(rating_matrix_batch, W1.T, items.T)
    z, logits, klrows = _tc_encoder(
        gathered, mask, W1b, b1.reshape(1, ENC_H), W2,
        b2.reshape(1, 2 * Z_DIM), itn, epsilon)
    kl = klrows[0, 0, 0]
    return z, logits, kl


# speedup vs baseline: 1.0426x; 1.0236x over previous
import functools

import jax
import jax.numpy as jnp
from jax import lax
from jax.experimental import pallas as pl
from jax.experimental.pallas import tpu as pltpu
from jax.experimental.pallas import tpu_sc as plsc

NUM_ITEMS = 8192
BATCH = 1024
HALF = BATCH // 2
ENC_H = 600
Z_DIM = 200
TAU = 0.2
EPS = 1e-12

_NC = 2
_NS = 16
_NW = _NC * _NS
_CH = 8

_BM = 128
_NBLK_H = HALF // _BM


def _sc_gather(gram, idx, nrows):
    rows_per_w = nrows // _NW
    nchunk = rows_per_w // _CH
    mesh = plsc.VectorSubcoreMesh(core_axis_name="c", subcore_axis_name="s")

    @functools.partial(
        pl.kernel,
        mesh=mesh,
        out_type=jax.ShapeDtypeStruct((nrows, NUM_ITEMS), jnp.float32),
        scratch_types=[
            pltpu.VMEM((rows_per_w,), jnp.int32),
            pltpu.VMEM((_CH, NUM_ITEMS), jnp.float32),
            pltpu.SemaphoreType.DMA,
        ],
    )
    def gather_kernel(table_hbm, idx_hbm, out_hbm, idx_v, buf, sem):
        wid = lax.axis_index("s") * _NC + lax.axis_index("c")
        base = wid * rows_per_w
        pltpu.sync_copy(idx_hbm.at[pl.ds(base, rows_per_w)], idx_v)
        for c in range(nchunk):
            pltpu.async_copy(
                table_hbm.at[idx_v.at[pl.ds(c * _CH, _CH)]], buf, sem).wait()
            pltpu.sync_copy(buf, out_hbm.at[pl.ds(base + c * _CH, _CH)])

    return gather_kernel(gram, idx)


def _prep_body(w1T_ref, itemsT_ref, w1b_ref, itn_ref):
    w1b_ref[...] = jnp.transpose(w1T_ref[...].astype(jnp.bfloat16))
    it = itemsT_ref[...]
    s = jnp.sqrt(jnp.sum(it * it, axis=0, keepdims=True))
    itn_ref[...] = (it / jnp.maximum(s, EPS)).astype(jnp.bfloat16)


def _tc_prep(W1T, itemsT):
    return pl.pallas_call(
        _prep_body,
        out_shape=[
            jax.ShapeDtypeStruct((NUM_ITEMS, ENC_H), jnp.bfloat16),
            jax.ShapeDtypeStruct((Z_DIM, NUM_ITEMS), jnp.bfloat16),
        ],
    )(W1T, itemsT)


def _make_tc_body(with_donors):
    def _tc_body(*refs):
        if with_donors:
            (gath_ref, rate_ref, w1b_ref, b1_ref, w2_ref, b2_ref, itn_ref,
             eps_ref, kl_in_ref, zd_ref, ld_ref,
             z_ref, logits_ref, klrow_ref, acc_ref) = refs
        else:
            (gath_ref, rate_ref, w1b_ref, b1_ref, w2_ref, b2_ref, itn_ref,
             eps_ref, kl_in_ref,
             z_ref, logits_ref, klrow_ref, acc_ref) = refs
        i = pl.program_id(0)

        vb = jnp.where(rate_ref[...] > 0, gath_ref[...], 0.0).astype(jnp.bfloat16)
        vf = vb.astype(jnp.float32)
        s1 = jnp.maximum(jnp.sum(vf, axis=1, keepdims=True), EPS)
        l2 = jnp.sqrt(jnp.sum(vf * vf, axis=1, keepdims=True)) / s1
        scale = 1.0 / (s1 * jnp.maximum(l2, EPS))

        pre = jnp.dot(vb, w1b_ref[...], preferred_element_type=jnp.float32)
        h = jnp.tanh(pre * scale + b1_ref[...])
        x2 = (jnp.dot(h, w2_ref[...], preferred_element_type=jnp.float32)
              + b2_ref[...])
        mean = x2[:, :Z_DIM]
        logvar = x2[:, Z_DIM:]
        std = jnp.exp(0.5 * logvar)
        z = mean + eps_ref[...] * std
        z_ref[...] = z

        zn = z * ((1.0 / TAU) /
                  jnp.maximum(jnp.sqrt(jnp.sum(z * z, axis=1, keepdims=True)),
                              EPS))
        logits_ref[...] = jnp.dot(zn.astype(jnp.bfloat16), itn_ref[...],
                                  preferred_element_type=jnp.float32)

        var = std * std
        klb = jnp.sum(mean * mean + var - 1.0 - logvar)
        prev = jnp.where(i == 0, 0.0, acc_ref[0, 0])
        total = prev + klb
        acc_ref[0, 0] = total
        klrow_ref[...] = (kl_in_ref[...]
                          + jnp.full((1, 1, 128), total * (0.5 / BATCH),
                                     jnp.float32))
    return _tc_body


def _tc_encoder_half(half, gathered, rating, W1b, b1, W2, b2, itn, epsilon,
                     kl_in, z_buf=None, logits_buf=None):
    off = half * _NBLK_H
    with_donors = z_buf is not None
    in_specs = [
        pl.BlockSpec((_BM, NUM_ITEMS), lambda i: (i, 0)),
        pl.BlockSpec((_BM, NUM_ITEMS), lambda i: (i + off, 0)),
        pl.BlockSpec((NUM_ITEMS, ENC_H), lambda i: (0, 0)),
        pl.BlockSpec((1, ENC_H), lambda i: (0, 0)),
        pl.BlockSpec((ENC_H, 2 * Z_DIM), lambda i: (0, 0)),
        pl.BlockSpec((1, 2 * Z_DIM), lambda i: (0, 0)),
        pl.BlockSpec((Z_DIM, NUM_ITEMS), lambda i: (0, 0)),
        pl.BlockSpec((_BM, Z_DIM), lambda i: (i + off, 0)),
        pl.BlockSpec((1, 1, 128), lambda i: (0, 0, 0)),
    ]
    args = [gathered, rating, W1b, b1, W2, b2, itn, epsilon, kl_in]
    aliases = {}
    if with_donors:
        in_specs += [
            pl.BlockSpec(memory_space=pl.ANY),
            pl.BlockSpec(memory_space=pl.ANY),
        ]
        args += [z_buf, logits_buf]
        aliases = {9: 0, 10: 1}
    return pl.pallas_call(
        _make_tc_body(with_donors),
        grid=(_NBLK_H,),
        in_specs=in_specs,
        out_specs=[
            pl.BlockSpec((_BM, Z_DIM), lambda i: (i + off, 0)),
            pl.BlockSpec((_BM, NUM_ITEMS), lambda i: (i + off, 0)),
            pl.BlockSpec((1, 1, 128), lambda i: (0, 0, 0)),
        ],
        out_shape=[
            jax.ShapeDtypeStruct((BATCH, Z_DIM), jnp.float32),
            jax.ShapeDtypeStruct((BATCH, NUM_ITEMS), jnp.float32),
            jax.ShapeDtypeStruct((1, 1, 128), jnp.float32),
        ],
        scratch_shapes=[
            pltpu.SMEM((1, 1), jnp.float32),
        ],
        input_output_aliases=aliases,
    )(*args)


def kernel(rating_matrix_batch, rating_matrix_batch2, gram_matrix, W1, b1, W2,
           b2, items, epsilon):
    idx = rating_matrix_batch2
    gathered_a = _sc_gather(gram_matrix, idx[:HALF], HALF)
    gathered_b = _sc_gather(gram_matrix, idx[HALF:], HALF)
    W1b, itn = _tc_prep(W1.T, items.T)
    b1r = b1.reshape(1, ENC_H)
    b2r = b2.reshape(1, 2 * Z_DIM)
    kl0 = jnp.zeros((1, 1, 128), jnp.float32)
    z1, lg1, kl1 = _tc_encoder_half(
        0, gathered_a, rating_matrix_batch, W1b, b1r, W2, b2r, itn, epsilon,
        kl0)
    z, logits, klrows = _tc_encoder_half(
        1, gathered_b, rating_matrix_batch, W1b, b1r, W2, b2r, itn, epsilon,
        kl1, z1, lg1)
    kl = klrows[0, 0, 0]
    return z, logits, kl


# speedup vs baseline: 1.0651x; 1.0216x over previous
import functools

import jax
import jax.numpy as jnp
from jax import lax
from jax.experimental import pallas as pl
from jax.experimental.pallas import tpu as pltpu
from jax.experimental.pallas import tpu_sc as plsc

NUM_ITEMS = 8192
BATCH = 1024
HALF = BATCH // 2
ENC_H = 600
Z_DIM = 200
TAU = 0.2
EPS = 1e-12

_NC = 2
_NS = 16
_NW = _NC * _NS
_CH = 8

_BM = 128
_NBLK_H = HALF // _BM


def _sc_gather(gram, idx, nrows):
    rows_per_w = nrows // _NW
    nchunk = rows_per_w // _CH
    mesh = plsc.VectorSubcoreMesh(core_axis_name="c", subcore_axis_name="s")

    @functools.partial(
        pl.kernel,
        mesh=mesh,
        out_type=jax.ShapeDtypeStruct((nrows, NUM_ITEMS), jnp.float32),
        scratch_types=[
            pltpu.VMEM((rows_per_w,), jnp.int32),
            pltpu.VMEM((_CH, NUM_ITEMS), jnp.float32),
            pltpu.SemaphoreType.DMA,
        ],
    )
    def gather_kernel(table_hbm, idx_hbm, out_hbm, idx_v, buf, sem):
        wid = lax.axis_index("s") * _NC + lax.axis_index("c")
        base = wid * rows_per_w
        pltpu.sync_copy(idx_hbm.at[pl.ds(base, rows_per_w)], idx_v)
        for c in range(nchunk):
            pltpu.async_copy(
                table_hbm.at[idx_v.at[pl.ds(c * _CH, _CH)]], buf, sem).wait()
            pltpu.sync_copy(buf, out_hbm.at[pl.ds(base + c * _CH, _CH)])

    return gather_kernel(gram, idx)


_KCH = 1024


def _prep_body(w1T_ref, itemsT_ref, w1b_ref, itn_ref):
    w1b_ref[...] = jnp.transpose(w1T_ref[...].astype(jnp.bfloat16))
    it = itemsT_ref[...]
    s = jnp.sqrt(jnp.sum(it * it, axis=0, keepdims=True))
    itn_ref[...] = (it / jnp.maximum(s, EPS)).astype(jnp.bfloat16)


def _tc_prep(W1T, itemsT):
    return pl.pallas_call(
        _prep_body,
        grid=(NUM_ITEMS // _KCH,),
        in_specs=[
            pl.BlockSpec((ENC_H, _KCH), lambda i: (0, i)),
            pl.BlockSpec((Z_DIM, _KCH), lambda i: (0, i)),
        ],
        out_specs=[
            pl.BlockSpec((_KCH, ENC_H), lambda i: (i, 0)),
            pl.BlockSpec((Z_DIM, _KCH), lambda i: (0, i)),
        ],
        out_shape=[
            jax.ShapeDtypeStruct((NUM_ITEMS, ENC_H), jnp.bfloat16),
            jax.ShapeDtypeStruct((Z_DIM, NUM_ITEMS), jnp.bfloat16),
        ],
    )(W1T, itemsT)


def _make_tc_body(with_donors):
    def _tc_body(*refs):
        if with_donors:
            (gath_ref, rate_ref, w1b_ref, b1_ref, w2_ref, b2_ref, itn_ref,
             eps_ref, kl_in_ref, zd_ref, ld_ref,
             z_ref, logits_ref, klrow_ref, acc_ref) = refs
        else:
            (gath_ref, rate_ref, w1b_ref, b1_ref, w2_ref, b2_ref, itn_ref,
             eps_ref, kl_in_ref,
             z_ref, logits_ref, klrow_ref, acc_ref) = refs
        i = pl.program_id(0)

        vb = jnp.where(rate_ref[...] > 0, gath_ref[...], 0.0).astype(jnp.bfloat16)
        vf = vb.astype(jnp.float32)
        s1 = jnp.maximum(jnp.sum(vf, axis=1, keepdims=True), EPS)
        l2 = jnp.sqrt(jnp.sum(vf * vf, axis=1, keepdims=True)) / s1
        scale = 1.0 / (s1 * jnp.maximum(l2, EPS))

        pre = jnp.dot(vb, w1b_ref[...], preferred_element_type=jnp.float32)
        h = jnp.tanh(pre * scale + b1_ref[...])
        x2 = (jnp.dot(h, w2_ref[...], preferred_element_type=jnp.float32)
              + b2_ref[...])
        mean = x2[:, :Z_DIM]
        logvar = x2[:, Z_DIM:]
        std = jnp.exp(0.5 * logvar)
        z = mean + eps_ref[...] * std
        z_ref[...] = z

        zn = z * ((1.0 / TAU) /
                  jnp.maximum(jnp.sqrt(jnp.sum(z * z, axis=1, keepdims=True)),
                              EPS))
        logits_ref[...] = jnp.dot(zn.astype(jnp.bfloat16), itn_ref[...],
                                  preferred_element_type=jnp.float32)

        var = std * std
        klb = jnp.sum(mean * mean + var - 1.0 - logvar)
        prev = jnp.where(i == 0, 0.0, acc_ref[0, 0])
        total = prev + klb
        acc_ref[0, 0] = total
        klrow_ref[...] = (kl_in_ref[...]
                          + jnp.full((1, 1, 128), total * (0.5 / BATCH),
                                     jnp.float32))
    return _tc_body


def _tc_encoder_half(half, gathered, rating, W1b, b1, W2, b2, itn, epsilon,
                     kl_in, z_buf=None, logits_buf=None):
    off = half * _NBLK_H
    with_donors = z_buf is not None
    in_specs = [
        pl.BlockSpec((_BM, NUM_ITEMS), lambda i: (i, 0)),
        pl.BlockSpec((_BM, NUM_ITEMS), lambda i: (i + off, 0)),
        pl.BlockSpec((NUM_ITEMS, ENC_H), lambda i: (0, 0)),
        pl.BlockSpec((1, ENC_H), lambda i: (0, 0)),
        pl.BlockSpec((ENC_H, 2 * Z_DIM), lambda i: (0, 0)),
        pl.BlockSpec((1, 2 * Z_DIM), lambda i: (0, 0)),
        pl.BlockSpec((Z_DIM, NUM_ITEMS), lambda i: (0, 0)),
        pl.BlockSpec((_BM, Z_DIM), lambda i: (i + off, 0)),
        pl.BlockSpec((1, 1, 128), lambda i: (0, 0, 0)),
    ]
    args = [gathered, rating, W1b, b1, W2, b2, itn, epsilon, kl_in]
    aliases = {}
    if with_donors:
        in_specs += [
            pl.BlockSpec(memory_space=pl.ANY),
            pl.BlockSpec(memory_space=pl.ANY),
        ]
        args += [z_buf, logits_buf]
        aliases = {9: 0, 10: 1}
    return pl.pallas_call(
        _make_tc_body(with_donors),
        grid=(_NBLK_H,),
        in_specs=in_specs,
        out_specs=[
            pl.BlockSpec((_BM, Z_DIM), lambda i: (i + off, 0)),
            pl.BlockSpec((_BM, NUM_ITEMS), lambda i: (i + off, 0)),
            pl.BlockSpec((1, 1, 128), lambda i: (0, 0, 0)),
        ],
        out_shape=[
            jax.ShapeDtypeStruct((BATCH, Z_DIM), jnp.float32),
            jax.ShapeDtypeStruct((BATCH, NUM_ITEMS), jnp.float32),
            jax.ShapeDtypeStruct((1, 1, 128), jnp.float32),
        ],
        scratch_shapes=[
            pltpu.SMEM((1, 1), jnp.float32),
        ],
        input_output_aliases=aliases,
    )(*args)


def kernel(rating_matrix_batch, rating_matrix_batch2, gram_matrix, W1, b1, W2,
           b2, items, epsilon):
    idx = rating_matrix_batch2
    gathered_a = _sc_gather(gram_matrix, idx[:HALF], HALF)
    gathered_b = _sc_gather(gram_matrix, idx[HALF:], HALF)
    W1b, itn = _tc_prep(W1.T, items.T)
    b1r = b1.reshape(1, ENC_H)
    b2r = b2.reshape(1, 2 * Z_DIM)
    kl0 = jnp.zeros((1, 1, 128), jnp.float32)
    z1, lg1, kl1 = _tc_encoder_half(
        0, gathered_a, rating_matrix_batch, W1b, b1r, W2, b2r, itn, epsilon,
        kl0)
    z, logits, klrows = _tc_encoder_half(
        1, gathered_b, rating_matrix_batch, W1b, b1r, W2, b2r, itn, epsilon,
        kl1, z1, lg1)
    kl = klrows[0, 0, 0]
    return z, logits, kl
